# baseline (speedup 1.0000x reference)
import jax
import jax.numpy as jnp
from jax import lax
from jax.experimental import pallas as pl
from jax.experimental.pallas import tpu as pltpu
from jax.experimental.pallas import tpu_sc as plsc

_B = 16384
_D = 64
_W = 7
_WW = _W * 128
_SUBS = 18
_NSUB = 32 * _SUBS
_NBND = 608


def _body(table_t, sorted_hbm, order_hbm, bounds_hbm, out_hbm,
          sorted_v, order_v, bounds_v, pos_row, out_v, idxbuf, staged_sh,
          ld_sem, sc_sem, gt_sem):
    sid = lax.axis_index("s")
    wid = sid * 2 + lax.axis_index("c")
    sbase = sid * (_D * _WW)
    pltpu.sync_copy(sorted_hbm, sorted_v.at[pl.ds(0, _B)])
    pltpu.sync_copy(order_hbm, order_v.at[pl.ds(0, _B)])
    pltpu.sync_copy(bounds_hbm, bounds_v)
    iota = lax.iota(jnp.int32, 16)
    dvs = [iota + 16 * k for k in range(4)]

    def bucket(s, _):
        gid = wid * _SUBS + s
        lo_s = gid * _WW
        copies = [
            pltpu.async_copy(
                table_t.at[d, pl.ds(lo_s, _WW)],
                staged_sh.at[pl.ds(sbase + d * _WW, _WW)],
                ld_sem,
            )
            for d in range(_D)
        ]
        bv = bounds_v[pl.ds(gid, 16)]
        b0 = bv[0]
        b1 = bv[1]
        for c in copies:
            c.wait()

        def group(q, _):
            base = b0 + q * 128

            def hloop(h, _):
                off = base + h * 16
                valid = (off + iota) < b1
                pos16 = jnp.where(valid, order_v[pl.ds(off, 16)], -1)
                pos_row[0, pl.ds(h * 16, 16)] = pos16
                wv = jnp.clip(sorted_v[pl.ds(off, 16)] - lo_s, 0, _WW - 1)
                gcopies = []
                for j in range(16):
                    wsp = jnp.full((16,), wv[j], jnp.int32)
                    for k in range(8):
                        ent = dvs[k] * _WW + wsp if k < 4 else wsp
                        idxbuf[j % 8, pl.ds(k * 16, 16)] = ent
                    gcopies.append(pltpu.async_copy(
                        staged_sh.at[pl.ds(sbase, _D * _WW)].at[idxbuf.at[j % 8]],
                        out_v.at[h * 16 + j],
                        gt_sem,
                    ))
                    if j % 8 == 7:
                        for c in gcopies:
                            c.wait()
                        gcopies = []
                return 0

            lax.fori_loop(0, 8, hloop, 0)
            pltpu.async_copy(
                out_v,
                out_hbm.at[plsc.Indices(pos_row.at[0], ignored_value=-1)],
                sc_sem,
            ).wait()
            return 0

        lax.fori_loop(0, (b1 - b0 + 127) // 128, group, 0)
        return 0

    lax.fori_loop(0, _SUBS, bucket, 0)


def kernel(batch, embedding_weight):
    table_t = embedding_weight.T
    idx = batch.astype(jnp.int32)
    order = jnp.argsort(idx).astype(jnp.int32)
    sorted_idx = jnp.take(idx, order)
    bounds = jnp.searchsorted(
        sorted_idx, jnp.arange(_NBND, dtype=jnp.int32) * _WW
    ).astype(jnp.int32)
    mesh = plsc.VectorSubcoreMesh(core_axis_name="c", subcore_axis_name="s")
    gather = pl.kernel(
        _body,
        mesh=mesh,
        out_type=jax.ShapeDtypeStruct((_B, 128), jnp.float32),
        scratch_types=[
            pltpu.VMEM((_B + 128,), jnp.int32),
            pltpu.VMEM((_B + 128,), jnp.int32),
            pltpu.VMEM((_NBND,), jnp.int32),
            pltpu.VMEM((1, 128), jnp.int32),
            pltpu.VMEM((128, 128), jnp.float32),
            pltpu.VMEM((8, 128), jnp.int32),
            pltpu.VMEM_SHARED((16 * _D * _WW,), jnp.float32),
            pltpu.SemaphoreType.DMA,
            pltpu.SemaphoreType.DMA,
            pltpu.SemaphoreType.DMA,
        ],
    )
    out = gather(table_t, sorted_idx, order, bounds)
    return out[:, :_D]


# speedup vs baseline: 7.9380x; 7.9380x over previous
import jax
import jax.numpy as jnp
from jax import lax
from jax.experimental import pallas as pl
from jax.experimental.pallas import tpu as pltpu
from jax.experimental.pallas import tpu_sc as plsc

_B = 16384
_D = 64
_W = 7
_WW = _W * 128
_SUBS = 18
_NSUB = 32 * _SUBS
_NBND = 608


def _body(table_t, sorted_hbm, order_hbm, bounds_hbm, out_hbm,
          sorted_v, order_v, bounds_v, pos_row, out_v, idxbuf, staged_sh,
          ld_sem, sc_sem, gt_sem):
    sid = lax.axis_index("s")
    wid = sid * 2 + lax.axis_index("c")
    sbase = sid * (_D * _WW)
    pltpu.sync_copy(sorted_hbm, sorted_v.at[pl.ds(0, _B)])
    pltpu.sync_copy(order_hbm, order_v.at[pl.ds(0, _B)])
    pltpu.sync_copy(bounds_hbm, bounds_v)
    iota = lax.iota(jnp.int32, 16)
    dvs = [iota + 16 * k for k in range(4)]

    def bucket(s, _):
        gid = wid * _SUBS + s
        lo_s = gid * _WW
        copies = [
            pltpu.async_copy(
                table_t.at[d, pl.ds(lo_s, _WW)],
                staged_sh.at[pl.ds(sbase + d * _WW, _WW)],
                ld_sem,
            )
            for d in range(_D)
        ]
        bv = bounds_v[pl.ds(gid, 16)]
        b0 = bv[0]
        b1 = bv[1]
        for c in copies:
            c.wait()

        def group(q, _):
            base = b0 + q * 128

            def fire(h):
                off = base + h * 16
                valid = (off + iota) < b1
                pos16 = jnp.where(valid, order_v[pl.ds(off, 16)], -1)
                pos_row[0, pl.ds(h * 16, 16)] = pos16
                wv = jnp.clip(sorted_v[pl.ds(off, 16)] - lo_s, 0, _WW - 1)
                cps = []
                for j in range(16):
                    r = h * 16 + j
                    wsp = jnp.full((16,), wv[j], jnp.int32)
                    for k in range(4):
                        idxbuf[r, pl.ds(k * 16, 16)] = dvs[k] * _WW + wsp
                    cps.append(pltpu.async_copy(
                        staged_sh.at[pl.ds(sbase, _D * _WW)].at[idxbuf.at[r]],
                        out_v.at[r, pl.ds(0, _D)],
                        gt_sem,
                    ))
                return cps

            def drain(cps):
                for c in cps:
                    c.wait()

            for h in range(8):

                @pl.when(base + h * 16 < b1)
                def _(h=h):
                    drain(fire(h))

                @pl.when(base + h * 16 >= b1)
                def _(h=h):
                    pos_row[0, pl.ds(h * 16, 16)] = jnp.full((16,), -1, jnp.int32)

            pltpu.async_copy(
                out_v,
                out_hbm.at[plsc.Indices(pos_row.at[0], ignored_value=-1)],
                sc_sem,
            ).wait()
            return 0

        lax.fori_loop(0, (b1 - b0 + 127) // 128, group, 0)
        return 0

    lax.fori_loop(0, _SUBS, bucket, 0)


def kernel(batch, embedding_weight):
    table_t = embedding_weight.T
    idx = batch.astype(jnp.int32)
    order = jnp.argsort(idx).astype(jnp.int32)
    sorted_idx = jnp.take(idx, order)
    bounds = jnp.searchsorted(
        sorted_idx, jnp.arange(_NBND, dtype=jnp.int32) * _WW
    ).astype(jnp.int32)
    mesh = plsc.VectorSubcoreMesh(core_axis_name="c", subcore_axis_name="s")
    gather = pl.kernel(
        _body,
        mesh=mesh,
        out_type=jax.ShapeDtypeStruct((_B, 128), jnp.float32),
        scratch_types=[
            pltpu.VMEM((_B + 128,), jnp.int32),
            pltpu.VMEM((_B + 128,), jnp.int32),
            pltpu.VMEM((_NBND,), jnp.int32),
            pltpu.VMEM((1, 128), jnp.int32),
            pltpu.VMEM((128, 128), jnp.float32),
            pltpu.VMEM((128, _D), jnp.int32),
            pltpu.VMEM_SHARED((16 * _D * _WW,), jnp.float32),
            pltpu.SemaphoreType.DMA,
            pltpu.SemaphoreType.DMA,
            pltpu.SemaphoreType.DMA,
        ],
    )
    out = gather(table_t, sorted_idx, order, bounds)
    return out[:, :_D]
